# baseline (speedup 1.0000x reference)
import functools

import jax
import jax.numpy as jnp
from jax import lax
from jax.experimental import pallas as pl
from jax.experimental.pallas import tpu as pltpu
from jax.experimental.pallas import tpu_sc as plsc

K = 2
TAU = 0.35
ALPHA_SELFLOOP = 0.5
EPS = 1e-06
LAMBDA_FUSE = 0.05

_M = 10
_CV = 100
_NT = 19200
_NW = 25
_CHUNK = 6400
_COLS = 256
_NB = _COLS // 16


def _cen_body(x_ref, cen_ref, mag_ref):
    f32 = jnp.float32
    for m in range(_M):
        g = x_ref[m]
        mag_ref[m] = jnp.sum(jnp.abs(g[0:75, :]), axis=0)
        for d in range(3):
            cen_ref[d * 10 + m] = jnp.mean(g[d * 25:(d + 1) * 25, :], axis=0)


def _rsqrt_nr(x):
    i = lax.bitcast_convert_type(x, jnp.int32)
    i = jnp.int32(0x5F3759DF) - lax.shift_right_logical(i, 1)
    y = lax.bitcast_convert_type(i, jnp.float32)
    for _ in range(3):
        y = y * (1.5 - 0.5 * x * y * y)
    return y


def _adj_sc_body(cen_hbm, mag_hbm, adj_hbm, cen_v, mag_v, adj_v):
    f32 = jnp.float32
    wid = lax.axis_index("s") * 2 + lax.axis_index("c")

    @pl.when(wid < _NW)
    def _work():
        base = wid * _COLS
        pltpu.sync_copy(cen_hbm.at[:, pl.ds(base, _COLS)], cen_v)
        pltpu.sync_copy(mag_hbm.at[:, pl.ds(base, _COLS)], mag_v)

        def batch(k, carry):
            off = k * 16
            sl = pl.ds(off, 16)
            pres = [jnp.where(mag_v[m, sl] > EPS, 1.0, 0.0).astype(f32)
                    for m in range(_M)]
            cen = [[cen_v[d * 10 + m, sl] for d in range(3)]
                   for m in range(_M)]
            dsq = [[None] * _M for _ in range(_M)]
            for i in range(_M):
                for j in range(i + 1, _M):
                    s = None
                    for d in range(3):
                        df = cen[i][d] - cen[j][d]
                        s = df * df if s is None else s + df * df
                    s = jnp.maximum(s, 1e-12)
                    s = jnp.where(pres[i] * pres[j] > 0, s, 1e12)
                    dsq[i][j] = s
                    dsq[j][i] = s
            bigv = jnp.full((16,), 3.0e38, f32)
            w1l, w2l, i1l, i2l = [], [], [], []
            for i in range(_M):
                cand = [j for j in range(_M) if j != i]
                d1 = dsq[i][cand[0]]
                i1 = jnp.full((16,), cand[0], jnp.int32)
                d2 = bigv
                i2 = jnp.full((16,), cand[0], jnp.int32)
                for j in cand[1:]:
                    dj = dsq[i][j]
                    jv = jnp.full((16,), j, jnp.int32)
                    t1 = dj < d1
                    t2 = dj < d2
                    d2 = jnp.where(t1, d1, jnp.where(t2, dj, d2))
                    i2 = jnp.where(t1, i1, jnp.where(t2, jv, i2))
                    d1 = jnp.where(t1, dj, d1)
                    i1 = jnp.where(t1, jv, i1)
                dd1 = d1 * _rsqrt_nr(d1)
                dd2 = d2 * _rsqrt_nr(d2)
                e1 = jnp.exp(dd1 * jnp.float32(-1.0 / TAU))
                e2 = jnp.exp(dd2 * jnp.float32(-1.0 / TAU))
                ssum = e1 + e2 + 1e-06
                w1l.append(e1 / ssum)
                w2l.append(e2 / ssum)
                i1l.append(i1)
                i2l.append(i2)
            adj = [[None] * _M for _ in range(_M)]
            for i in range(_M):
                for j in range(_M):
                    a = (jnp.where(i1l[i] == j, w1l[i], 0.0)
                         + jnp.where(i2l[i] == j, w2l[i], 0.0))
                    adj[i][j] = a
            for i in range(_M):
                for j in range(i, _M):
                    v = 0.5 * (adj[i][j] + adj[j][i])
                    if i == j:
                        v = v + ALPHA_SELFLOOP
                    adj[i][j] = v
                    adj[j][i] = v
            dinv = []
            for i in range(_M):
                rs = adj[i][0]
                for j in range(1, _M):
                    rs = rs + adj[i][j]
                inv = 1.0 / (rs + 1e-06)
                for j in range(_M):
                    adj[i][j] = adj[i][j] * inv
                deg = jnp.maximum(rs * inv, 1e-06)
                dinv.append(_rsqrt_nr(deg))
            for i in range(_M):
                for j in range(_M):
                    adj_v[j, i, sl] = adj[i][j] * dinv[i] * dinv[j]
            return carry

        lax.fori_loop(0, _NB, batch, 0)
        pltpu.sync_copy(adj_v, adj_hbm.at[:, :, pl.ds(base, _COLS)])


def _apply_body(x_ref, adj_ref, lam_ref, o_ref):
    f32 = jnp.float32
    lam = lam_ref[0, 0]
    adjb = adj_ref[...].astype(jnp.bfloat16)
    xb = [x_ref[j].astype(jnp.bfloat16) for j in range(_M)]
    for i in range(_M):
        acc = None
        for j in range(_M):
            t = adjb[j, i:i + 1, :] * xb[j]
            acc = t if acc is None else acc + t
        o_ref[i] = x_ref[i] + lam * acc.astype(f32)


def kernel(x, lam):
    N, C, T, V, M = x.shape
    NT = N * T
    B = 384
    xt = jnp.transpose(x, (4, 1, 3, 0, 2)).reshape(M, C * V, NT)
    lam2 = jnp.asarray(lam, jnp.float32).reshape(1, 1)

    cen, mag = pl.pallas_call(
        _cen_body,
        grid=(NT // B,),
        in_specs=[pl.BlockSpec((M, C * V, B), lambda i: (0, 0, i))],
        out_specs=[
            pl.BlockSpec((30, B), lambda i: (0, i)),
            pl.BlockSpec((10, B), lambda i: (0, i)),
        ],
        out_shape=[
            jax.ShapeDtypeStruct((30, NT), jnp.float32),
            jax.ShapeDtypeStruct((10, NT), jnp.float32),
        ],
    )(xt)

    mesh = plsc.VectorSubcoreMesh(core_axis_name="c", subcore_axis_name="s")
    adj_fn = functools.partial(
        pl.kernel,
        mesh=mesh,
        out_type=pltpu.HBM((_M, _M, _CHUNK), jnp.float32),
        scratch_types=[
            pltpu.VMEM((30, _COLS), jnp.float32),
            pltpu.VMEM((10, _COLS), jnp.float32),
            pltpu.VMEM((_M, _M, _COLS), jnp.float32),
        ],
    )(_adj_sc_body)
    adjT = jnp.concatenate(
        [adj_fn(lax.slice_in_dim(cen, c, c + _CHUNK, axis=1),
                lax.slice_in_dim(mag, c, c + _CHUNK, axis=1))
         for c in range(0, NT, _CHUNK)], axis=2)

    out3 = pl.pallas_call(
        _apply_body,
        grid=(NT // B,),
        in_specs=[
            pl.BlockSpec((M, C * V, B), lambda i: (0, 0, i)),
            pl.BlockSpec((M, M, B), lambda i: (0, 0, i)),
            pl.BlockSpec((1, 1), lambda i: (0, 0)),
        ],
        out_specs=pl.BlockSpec((M, C * V, B), lambda i: (0, 0, i)),
        out_shape=jax.ShapeDtypeStruct((M, C * V, NT), jnp.float32),
    )(xt, adjT, lam2)
    out = out3.reshape(M, C, V, N, T).transpose(3, 1, 4, 2, 0)
    return out


# speedup vs baseline: 1.2457x; 1.2457x over previous
import jax
import jax.numpy as jnp
from jax.experimental import pallas as pl

K = 2
TAU = 0.35
ALPHA_SELFLOOP = 0.5
EPS = 1e-06
LAMBDA_FUSE = 0.05

_M = 10
_CV = 100


def _taglayer_body(x_ref, lam_ref, o_ref):
    B = x_ref.shape[-1]
    f32 = jnp.float32

    cen = []
    mag = []
    cen_d = [[], [], []]
    for m in range(_M):
        g = x_ref[m][0:75, :].astype(f32)
        mag.append(jnp.sum(jnp.abs(g), axis=0))
        for d in range(3):
            cen_d[d].append(jnp.mean(g[d * 25:(d + 1) * 25, :], axis=0))
    presf = (jnp.stack(mag, axis=0) > EPS).astype(f32)

    dsq = jnp.zeros((_M, _M, B), dtype=f32)
    for d in range(3):
        cd = jnp.stack(cen_d[d], axis=0)
        diff = cd[:, None, :] - cd[None, :, :]
        dsq = dsq + diff * diff
    dist = jnp.sqrt(jnp.clip(dsq, 1e-12, None))
    pair_ok = presf[:, None, :] * presf[None, :, :] > 0
    dist = jnp.where(pair_ok, dist, 1000000.0)
    ii = jax.lax.broadcasted_iota(jnp.int32, (_M, _M, B), 0)
    jj = jax.lax.broadcasted_iota(jnp.int32, (_M, _M, B), 1)
    eye3 = (ii == jj).astype(f32)
    dist = dist + eye3 * 1000000.0

    d1 = dist[0]
    i1 = jnp.zeros((_M, B), dtype=jnp.int32)
    for j in range(1, _M):
        dj = dist[j]
        take = dj < d1
        d1 = jnp.where(take, dj, d1)
        i1 = jnp.where(take, j, i1)
    big = jnp.float32(3.0e38)
    d2 = jnp.where(i1 == 0, big, dist[0])
    i2 = jnp.zeros((_M, B), dtype=jnp.int32)
    for j in range(1, _M):
        dj = dist[j]
        take = jnp.logical_and(i1 != j, dj < d2)
        d2 = jnp.where(take, dj, d2)
        i2 = jnp.where(take, j, i2)

    w1 = jnp.exp(-d1 / TAU)
    w2 = jnp.exp(-d2 / TAU)
    s = w1 + w2 + 1e-06
    w1 = w1 / s
    w2 = w2 / s
    cols = []
    for j in range(_M):
        cols.append(w1 * (i1 == j).astype(f32) + w2 * (i2 == j).astype(f32))
    adjT = jnp.stack(cols, axis=0)

    adjT = 0.5 * (adjT + jnp.transpose(adjT, (1, 0, 2)))
    adjT = adjT + ALPHA_SELFLOOP * eye3
    rs = jnp.sum(adjT, axis=0)
    adjT = adjT / (rs + 1e-06)[None, :, :]
    deg = jnp.clip(jnp.sum(adjT, axis=0), 1e-06, None)
    dinv = jax.lax.rsqrt(deg)
    dinv_j = dinv[:, None, :]
    adjT = adjT * dinv[None, :, :] * dinv_j

    lamb = lam_ref[0, 0].astype(jnp.bfloat16)
    adjb = adjT.astype(jnp.bfloat16)
    for i in range(_M):
        acc = None
        for j in range(_M):
            t = adjb[j, i:i + 1, :] * x_ref[j]
            acc = t if acc is None else acc + t
        o_ref[i] = x_ref[i] + lamb * acc


def kernel(x, lam):
    N, C, T, V, M = x.shape
    NT = N * T
    B = 384
    xt = jnp.transpose(x.astype(jnp.bfloat16), (4, 1, 3, 0, 2))
    xt = xt.reshape(M, C * V, NT)
    lam2 = jnp.asarray(lam, jnp.float32).reshape(1, 1)
    out3 = pl.pallas_call(
        _taglayer_body,
        grid=(NT // B,),
        in_specs=[
            pl.BlockSpec((M, C * V, B), lambda i: (0, 0, i)),
            pl.BlockSpec((1, 1), lambda i: (0, 0)),
        ],
        out_specs=pl.BlockSpec((M, C * V, B), lambda i: (0, 0, i)),
        out_shape=jax.ShapeDtypeStruct((M, C * V, NT), jnp.bfloat16),
    )(xt, lam2)
    out = out3.reshape(M, C, V, N, T).transpose(3, 1, 4, 2, 0)
    return out.astype(jnp.float32)
